# Initial kernel scaffold; baseline (speedup 1.0000x reference)
import jax
import jax.numpy as jnp
from jax.experimental import pallas as pl


def kernel(label_ids, pos_ids, dep_ids, label_table, pos_table, dep_table):
    raise NotImplementedError("write your pallas kernel here")


# speedup vs baseline: 3.2861x; 3.2861x over previous
import functools

import jax
import jax.numpy as jnp
from jax import lax
from jax.experimental import pallas as pl
from jax.experimental.pallas import tpu as pltpu
from jax.experimental.pallas import tpu_sc as plsc

_B, _L = 4096, 200
_N = _B * _L
_NC, _NS = 2, 16
_NW = _NC * _NS
_PER_W = _N // _NW
_CHUNK = 512
_NCHUNK = _PER_W // _CHUNK

_LABEL_DIM, _POS_DIM, _DEP_DIM = 64, 32, 32


def _build():
    mesh = plsc.VectorSubcoreMesh(core_axis_name="c", subcore_axis_name="s")

    @functools.partial(
        pl.kernel,
        out_type=(
            jax.ShapeDtypeStruct((_N, _POS_DIM), jnp.float32),
            jax.ShapeDtypeStruct((_N, _DEP_DIM), jnp.float32),
            jax.ShapeDtypeStruct((_N, _LABEL_DIM), jnp.float32),
        ),
        mesh=mesh,
        compiler_params=pltpu.CompilerParams(use_tc_tiling_on_sc=False),
        scratch_types=(
            pltpu.VMEM((_CHUNK,), jnp.int32),
            pltpu.VMEM((_CHUNK, _POS_DIM), jnp.float32),
            pltpu.VMEM((_CHUNK,), jnp.int32),
            pltpu.VMEM((_CHUNK, _DEP_DIM), jnp.float32),
            pltpu.VMEM((_CHUNK,), jnp.int32),
            pltpu.VMEM((_CHUNK, _LABEL_DIM), jnp.float32),
            pltpu.SemaphoreType.DMA,
        ),
    )
    def emb_kernel(label_ids, pos_ids, dep_ids, label_tab, pos_tab, dep_tab,
                   pos_out, dep_out, label_out,
                   pos_idx, pos_rows, dep_idx, dep_rows, lab_idx, lab_rows,
                   sem):
        wid = lax.axis_index("s") * _NC + lax.axis_index("c")
        w_base = wid * _PER_W

        def body(i, carry):
            sl = pl.ds(w_base + i * _CHUNK, _CHUNK)
            pltpu.sync_copy(pos_ids.at[sl], pos_idx)
            pltpu.async_copy(pos_tab.at[pos_idx], pos_rows, sem).wait()
            pltpu.sync_copy(pos_rows, pos_out.at[sl])
            pltpu.sync_copy(dep_ids.at[sl], dep_idx)
            pltpu.async_copy(dep_tab.at[dep_idx], dep_rows, sem).wait()
            pltpu.sync_copy(dep_rows, dep_out.at[sl])
            pltpu.sync_copy(label_ids.at[sl], lab_idx)
            pltpu.async_copy(label_tab.at[lab_idx], lab_rows, sem).wait()
            pltpu.sync_copy(lab_rows, label_out.at[sl])
            return carry

        lax.fori_loop(0, _NCHUNK, body, 0)

    return emb_kernel


_EMB = _build()


def kernel(label_ids, pos_ids, dep_ids, label_table, pos_table, dep_table):
    lab = label_ids.reshape(_N).astype(jnp.int32)
    pos = pos_ids.reshape(_N).astype(jnp.int32)
    dep = dep_ids.reshape(_N).astype(jnp.int32)
    pos_e, dep_e, lab_e = _EMB(lab, pos, dep, label_table, pos_table, dep_table)
    return (
        pos_e.reshape(_B, _L, _POS_DIM),
        dep_e.reshape(_B, _L, _DEP_DIM),
        lab_e.reshape(_B, _L, _LABEL_DIM),
    )


# speedup vs baseline: 3.3804x; 1.0287x over previous
import functools

import jax
import jax.numpy as jnp
from jax import lax
from jax.experimental import pallas as pl
from jax.experimental.pallas import tpu as pltpu
from jax.experimental.pallas import tpu_sc as plsc

_B, _L = 4096, 200
_N = _B * _L
_NC, _NS = 2, 16
_NW = _NC * _NS
_PER_W = _N // _NW
_CHUNK = 400
_NCHUNK = _PER_W // _CHUNK

_LABEL_DIM, _POS_DIM, _DEP_DIM = 64, 32, 32


def _build():
    mesh = plsc.VectorSubcoreMesh(core_axis_name="c", subcore_axis_name="s")

    @functools.partial(
        pl.kernel,
        out_type=(
            jax.ShapeDtypeStruct((_N, _POS_DIM), jnp.float32),
            jax.ShapeDtypeStruct((_N, _DEP_DIM), jnp.float32),
            jax.ShapeDtypeStruct((_N, _LABEL_DIM), jnp.float32),
        ),
        mesh=mesh,
        compiler_params=pltpu.CompilerParams(use_tc_tiling_on_sc=False),
        scratch_types=(
            [pltpu.VMEM((_CHUNK,), jnp.int32) for _ in range(2)],
            [pltpu.VMEM((_CHUNK, _POS_DIM), jnp.float32) for _ in range(2)],
            [pltpu.VMEM((_CHUNK,), jnp.int32) for _ in range(2)],
            [pltpu.VMEM((_CHUNK, _DEP_DIM), jnp.float32) for _ in range(2)],
            [pltpu.VMEM((_CHUNK,), jnp.int32) for _ in range(2)],
            [pltpu.VMEM((_CHUNK, _LABEL_DIM), jnp.float32) for _ in range(2)],
            [pltpu.SemaphoreType.DMA for _ in range(3)],
            [pltpu.SemaphoreType.DMA for _ in range(3)],
            [pltpu.SemaphoreType.DMA for _ in range(3)],
        ),
    )
    def emb_kernel(label_ids, pos_ids, dep_ids, label_tab, pos_tab, dep_tab,
                   pos_out, dep_out, label_out,
                   pos_idx, pos_rows, dep_idx, dep_rows, lab_idx, lab_rows,
                   sl, sg, ss):
        wid = lax.axis_index("s") * _NC + lax.axis_index("c")
        w_base = wid * _PER_W

        tables = (
            (pos_ids, pos_tab, pos_out, pos_idx, pos_rows, 0),
            (dep_ids, dep_tab, dep_out, dep_idx, dep_rows, 1),
            (label_ids, label_tab, label_out, lab_idx, lab_rows, 2),
        )

        def chunk_slice(i):
            return pl.ds(w_base + i * _CHUNK, _CHUNK)

        def step(i, b, first, second):
            nb = 1 - b
            for ids, tab, out, idx, rows, t in tables:
                if not (first or second):
                    pltpu.make_async_copy(rows[b], out.at[chunk_slice(i)], ss[t]).wait()
                if not first:
                    pltpu.make_async_copy(tab.at[idx[nb]], rows[nb], sg[t]).wait()
                    pltpu.make_async_copy(rows[nb], out.at[chunk_slice(i - 1)],
                                          ss[t]).start()
                pltpu.make_async_copy(ids.at[chunk_slice(i)], idx[b], sl[t]).wait()
                pltpu.make_async_copy(tab.at[idx[b]], rows[b], sg[t]).start()
                if isinstance(i, int):
                    if i < _NCHUNK - 1:
                        pltpu.make_async_copy(ids.at[chunk_slice(i + 1)], idx[nb],
                                              sl[t]).start()
                else:
                    @pl.when(i < _NCHUNK - 1)
                    def _():
                        pltpu.make_async_copy(ids.at[chunk_slice(i + 1)], idx[nb],
                                              sl[t]).start()

        for ids, tab, out, idx, rows, t in tables:
            pltpu.make_async_copy(ids.at[chunk_slice(0)], idx[0], sl[t]).start()

        step(0, 0, True, False)
        step(1, 1, False, True)

        def body(i2, carry):
            step(2 * i2, 0, False, False)
            step(2 * i2 + 1, 1, False, False)
            return carry

        lax.fori_loop(1, _NCHUNK // 2, body, 0)

        last = _NCHUNK - 1
        for ids, tab, out, idx, rows, t in tables:
            pltpu.make_async_copy(tab.at[idx[1]], rows[1], sg[t]).wait()
            pltpu.make_async_copy(rows[1], out.at[chunk_slice(last)], ss[t]).start()
            pltpu.make_async_copy(rows[0], out.at[chunk_slice(last - 1)], ss[t]).wait()
            pltpu.make_async_copy(rows[1], out.at[chunk_slice(last)], ss[t]).wait()

    return emb_kernel


_EMB = _build()


def kernel(label_ids, pos_ids, dep_ids, label_table, pos_table, dep_table):
    lab = label_ids.reshape(_N).astype(jnp.int32)
    pos = pos_ids.reshape(_N).astype(jnp.int32)
    dep = dep_ids.reshape(_N).astype(jnp.int32)
    pos_e, dep_e, lab_e = _EMB(lab, pos, dep, label_table, pos_table, dep_table)
    return (
        pos_e.reshape(_B, _L, _POS_DIM),
        dep_e.reshape(_B, _L, _DEP_DIM),
        lab_e.reshape(_B, _L, _LABEL_DIM),
    )
